# baseline (speedup 1.0000x reference)
import functools

import jax
import jax.numpy as jnp
from jax import lax
from jax.experimental import pallas as pl
from jax.experimental.pallas import tpu as pltpu
from jax.experimental.pallas import tpu_sc as plsc

_LANES = 16


def _build(batch, maxlen, vocab, dim, half):
    info = plsc.get_sparse_core_info()
    nc, ns = info.num_cores, info.num_subcores
    nw = nc * ns
    n_half = maxlen // half
    total_chunks = batch * n_half
    chunks_per_w = total_chunks // nw
    vregs_per_row = dim // _LANES

    mesh = plsc.VectorSubcoreMesh(core_axis_name="c", subcore_axis_name="s")

    @functools.partial(
        pl.kernel,
        out_type=jax.ShapeDtypeStruct((total_chunks * half, dim), jnp.float32),
        mesh=mesh,
        scratch_types=[
            pltpu.VMEM((chunks_per_w, half), jnp.int32),
            pltpu.VMEM((maxlen, dim), jnp.float32),
            pltpu.VMEM((half, dim), jnp.float32),
            pltpu.SemaphoreType.DMA,
        ],
        compiler_params=pltpu.CompilerParams(use_tc_tiling_on_sc=False),
    )
    def emb(x_hbm, tok_hbm, pos_hbm, out_hbm, idx_v, pos_v, gbuf, sem):
        wid = lax.axis_index("s") * nc + lax.axis_index("c")
        chunk0 = wid * chunks_per_w
        pltpu.sync_copy(pos_hbm, pos_v)
        pltpu.sync_copy(x_hbm.at[pl.ds(chunk0, chunks_per_w)], idx_v)

        def do_pair(pair, _):
            for parity in range(n_half):
                c = pair * n_half + parity
                pltpu.async_copy(tok_hbm.at[idx_v.at[c]], gbuf, sem).wait()

                def add_row(r, _, parity=parity):
                    for q in range(vregs_per_row):
                        sl = pl.ds(q * _LANES, _LANES)
                        gbuf[r, sl] = gbuf[r, sl] + pos_v[parity * half + r, sl]
                    return ()

                lax.fori_loop(0, half, add_row, (), unroll=2)
                pltpu.sync_copy(
                    gbuf, out_hbm.at[pl.ds((chunk0 + c) * half, half)]
                )
            return ()

        lax.fori_loop(0, chunks_per_w // n_half, do_pair, ())

    return emb


def kernel(x, token_table, pos_table):
    batch, maxlen = x.shape
    vocab, dim = token_table.shape
    half = maxlen // 2
    x2 = x.astype(jnp.int32).reshape(batch * 2, half)
    emb = _build(batch, maxlen, vocab, dim, half)
    out = emb(x2, token_table, pos_table)
    return out.reshape(batch, maxlen, dim)


# speedup vs baseline: 1.1595x; 1.1595x over previous
import functools

import jax
import jax.numpy as jnp
from jax import lax
from jax.experimental import pallas as pl
from jax.experimental.pallas import tpu as pltpu
from jax.experimental.pallas import tpu_sc as plsc

_LANES = 16
_NBUF = 4


def _build(batch, maxlen, vocab, dim, half):
    info = plsc.get_sparse_core_info()
    nc, ns = info.num_cores, info.num_subcores
    nw = nc * ns
    n_half = maxlen // half
    total_chunks = batch * n_half
    chunks_per_w = total_chunks // nw
    vregs_per_row = dim // _LANES
    n_rounds = chunks_per_w // _NBUF

    mesh = plsc.VectorSubcoreMesh(core_axis_name="c", subcore_axis_name="s")

    @functools.partial(
        pl.kernel,
        out_type=jax.ShapeDtypeStruct((total_chunks * half, dim), jnp.float32),
        mesh=mesh,
        scratch_types=[
            pltpu.VMEM((chunks_per_w, half), jnp.int32),
            pltpu.VMEM((maxlen, dim), jnp.float32),
            [pltpu.VMEM((half, dim), jnp.float32)] * _NBUF,
            [pltpu.VMEM((half, dim), jnp.float32)] * _NBUF,
            pltpu.SemaphoreType.DMA((_NBUF,)),
            pltpu.SemaphoreType.DMA((_NBUF,)),
        ],
        compiler_params=pltpu.CompilerParams(use_tc_tiling_on_sc=False),
    )
    def emb(x_hbm, tok_hbm, pos_hbm, out_hbm, idx_v, pos_v, gbufs, obufs,
            gsem, osem):
        wid = lax.axis_index("s") * nc + lax.axis_index("c")
        chunk0 = wid * chunks_per_w
        pltpu.sync_copy(pos_hbm, pos_v)
        pltpu.sync_copy(x_hbm.at[pl.ds(chunk0, chunks_per_w)], idx_v)

        def gather(c, b):
            return pltpu.make_async_copy(
                tok_hbm.at[idx_v.at[c]], gbufs[b], gsem.at[b])

        def write(c, b):
            return pltpu.make_async_copy(
                obufs[b], out_hbm.at[pl.ds((chunk0 + c) * half, half)],
                osem.at[b])

        for b in range(_NBUF):
            gather(b, b).start()

        def do_round(g, _):
            for b in range(_NBUF):
                c = g * _NBUF + b
                gather(c, b).wait()

                @pl.when(g >= 1)
                def _():
                    write(c - _NBUF, b).wait()

                parity = b % 2

                def add_row(r, _, parity=parity, b=b):
                    for q in range(vregs_per_row):
                        sl = pl.ds(q * _LANES, _LANES)
                        obufs[b][r, sl] = (
                            gbufs[b][r, sl] + pos_v[parity * half + r, sl])
                    return ()

                lax.fori_loop(0, half, add_row, (), unroll=2)
                write(c, b).start()

                @pl.when(g < n_rounds - 1)
                def _():
                    gather(c + _NBUF, b).start()
            return ()

        lax.fori_loop(0, n_rounds, do_round, ())

        for b in range(_NBUF):
            write((n_rounds - 1) * _NBUF + b, b).wait()

    return emb


def kernel(x, token_table, pos_table):
    batch, maxlen = x.shape
    vocab, dim = token_table.shape
    half = maxlen // 2
    x2 = x.astype(jnp.int32).reshape(batch * 2, half)
    emb = _build(batch, maxlen, vocab, dim, half)
    out = emb(x2, token_table, pos_table)
    return out.reshape(batch, maxlen, dim)
